# baseline (speedup 1.0000x reference)
import functools

import jax
import jax.numpy as jnp
from jax import lax
from jax.experimental import pallas as pl
from jax.experimental.pallas import tpu as pltpu

_NBUF = 2
_CB = 4096
_NFULL = 24


def _stream_body(t_ref, x_hbm, loss_ref, bufs, tbuf, m_ref, s_ref, p_ref,
                 sems, tsem, *, n_rows, v_total):
    neg_inf = jnp.float32(-jnp.inf)
    tail_cols = v_total - _NFULL * _CB

    m_ref[...] = jnp.full(m_ref.shape, neg_inf, m_ref.dtype)
    s_ref[...] = jnp.zeros(s_ref.shape, s_ref.dtype)
    p_ref[...] = jnp.zeros(p_ref.shape, p_ref.dtype)
    t = t_ref[...]

    def copy(c, b):
        return pltpu.make_async_copy(
            x_hbm.at[:, pl.ds(c * _CB, _CB)], bufs.at[b], sems.at[b])

    for b in range(_NBUF):
        copy(jnp.int32(b), b).start()
    pltpu.make_async_copy(x_hbm.at[:, pl.ds(_NFULL * _CB, tail_cols)],
                          tbuf, tsem).start()

    def block_update(x, col):
        m_old = m_ref[...]
        m_new = jnp.maximum(m_old, jnp.max(x, axis=1, keepdims=True))
        s_ref[...] = (s_ref[...] * jnp.exp(m_old - m_new) +
                      jnp.sum(jnp.exp(x - m_new), axis=1, keepdims=True))
        p_ref[...] += jnp.sum(jnp.where(col == t, x, 0.0), axis=1,
                              keepdims=True)
        m_ref[...] = m_new

    def group(g, _):
        for b in range(_NBUF):
            c = g * _NBUF + b
            copy(c, b).wait()
            x = bufs[b, :, :]
            col = (lax.broadcasted_iota(jnp.int32, x.shape, 1) + c * _CB)
            block_update(x, col)

            @pl.when(c + _NBUF < _NFULL)
            def _():
                copy(c + _NBUF, b).start()
        return 0

    lax.fori_loop(0, _NFULL // _NBUF, group, 0)

    pltpu.make_async_copy(x_hbm.at[:, pl.ds(_NFULL * _CB, tail_cols)],
                          tbuf, tsem).wait()
    xt = tbuf[...]
    colt = (lax.broadcasted_iota(jnp.int32, xt.shape, 1) + _NFULL * _CB)
    xt = jnp.where(colt < v_total, xt, neg_inf)
    block_update(xt, colt)

    loss_ref[...] = m_ref[...] + jnp.log(s_ref[...]) - p_ref[...]


def _topk_body(loss_ref, out_ref, *, k):
    loss = loss_ref[...]
    b = lax.bitcast_convert_type(loss, jnp.int32)
    key = b ^ (lax.shift_right_arithmetic(b, 31) & jnp.int32(0x7FFFFFFF))

    def cnt_ge(thresh):
        return jnp.sum((key >= thresh).astype(jnp.int32))

    base0 = jnp.where(cnt_ge(jnp.int32(0)) >= k, jnp.int32(0),
                      jnp.int32(-(2**31)))

    def body(i, base):
        cand = base | lax.shift_left(jnp.int32(1), 30 - i)
        return jnp.where(cnt_ge(cand) >= k, cand, base)

    big_t = lax.fori_loop(0, 31, body, base0)
    tb = big_t ^ (lax.shift_right_arithmetic(big_t, 31) & jnp.int32(0x7FFFFFFF))
    tval = lax.bitcast_convert_type(tb, jnp.float32)
    gt = loss > tval
    cnt_gt = jnp.sum(gt.astype(jnp.float32))
    sum_gt = jnp.sum(jnp.where(gt, loss, 0.0))
    res = (sum_gt + (jnp.float32(k) - cnt_gt) * tval) / jnp.float32(k)
    out_ref[...] = jnp.full((1, 1), res, jnp.float32)


@jax.jit
def kernel(inputs, targets):
    n, v = inputs.shape
    k = int(0.25 * n)
    t2 = targets.reshape(n, 1).astype(jnp.int32)
    tail_cols = v - _NFULL * _CB
    loss = pl.pallas_call(
        functools.partial(_stream_body, n_rows=n, v_total=v),
        in_specs=[
            pl.BlockSpec((n, 1), lambda: (0, 0)),
            pl.BlockSpec(memory_space=pl.ANY),
        ],
        out_specs=pl.BlockSpec((n, 1), lambda: (0, 0)),
        out_shape=jax.ShapeDtypeStruct((n, 1), jnp.float32),
        scratch_shapes=[
            pltpu.VMEM((_NBUF, n, _CB), jnp.float32),
            pltpu.VMEM((n, tail_cols), jnp.float32),
            pltpu.VMEM((n, 1), jnp.float32),
            pltpu.VMEM((n, 1), jnp.float32),
            pltpu.VMEM((n, 1), jnp.float32),
            pltpu.SemaphoreType.DMA((_NBUF,)),
            pltpu.SemaphoreType.DMA,
        ],
    )(t2, inputs)
    loss8 = loss.reshape(8, n // 8)
    out = pl.pallas_call(
        functools.partial(_topk_body, k=k),
        out_shape=jax.ShapeDtypeStruct((1, 1), jnp.float32),
    )(loss8)
    return out[0, 0]


# speedup vs baseline: 1.0234x; 1.0234x over previous
import functools

import jax
import jax.numpy as jnp
from jax import lax
from jax.experimental import pallas as pl
from jax.experimental.pallas import tpu as pltpu

_NBUF = 4
_CB = 2048
_NFULL = 48


def _stream_body(t_ref, x_hbm, loss_ref, bufs, tbuf, m_ref, s_ref, p_ref,
                 sems, tsem, *, n_rows, v_total):
    neg_inf = jnp.float32(-jnp.inf)
    tail_cols = v_total - _NFULL * _CB

    m_ref[...] = jnp.full(m_ref.shape, neg_inf, m_ref.dtype)
    s_ref[...] = jnp.zeros(s_ref.shape, s_ref.dtype)
    p_ref[...] = jnp.zeros(p_ref.shape, p_ref.dtype)
    t = t_ref[...]

    def copy(c, b):
        return pltpu.make_async_copy(
            x_hbm.at[:, pl.ds(c * _CB, _CB)], bufs.at[b], sems.at[b])

    for b in range(_NBUF):
        copy(jnp.int32(b), b).start()
    pltpu.make_async_copy(x_hbm.at[:, pl.ds(_NFULL * _CB, tail_cols)],
                          tbuf, tsem).start()

    def block_update(x, col):
        m_old = m_ref[...]
        m_new = jnp.maximum(m_old, jnp.max(x, axis=1, keepdims=True))
        s_ref[...] = (s_ref[...] * jnp.exp(m_old - m_new) +
                      jnp.sum(jnp.exp(x - m_new), axis=1, keepdims=True))
        p_ref[...] += jnp.sum(jnp.where(col == t, x, 0.0), axis=1,
                              keepdims=True)
        m_ref[...] = m_new

    def group(g, _):
        for b in range(_NBUF):
            c = g * _NBUF + b
            copy(c, b).wait()
            x = bufs[b, :, :]
            col = (lax.broadcasted_iota(jnp.int32, x.shape, 1) + c * _CB)
            block_update(x, col)

            @pl.when(c + _NBUF < _NFULL)
            def _():
                copy(c + _NBUF, b).start()
        return 0

    lax.fori_loop(0, _NFULL // _NBUF, group, 0)

    pltpu.make_async_copy(x_hbm.at[:, pl.ds(_NFULL * _CB, tail_cols)],
                          tbuf, tsem).wait()
    xt = tbuf[...]
    colt = (lax.broadcasted_iota(jnp.int32, xt.shape, 1) + _NFULL * _CB)
    xt = jnp.where(colt < v_total, xt, neg_inf)
    block_update(xt, colt)

    loss_ref[...] = m_ref[...] + jnp.log(s_ref[...]) - p_ref[...]


def _topk_body(loss_ref, out_ref, *, k):
    loss = loss_ref[...]
    b = lax.bitcast_convert_type(loss, jnp.int32)
    key = b ^ (lax.shift_right_arithmetic(b, 31) & jnp.int32(0x7FFFFFFF))

    def cnt_ge(thresh):
        return jnp.sum((key >= thresh).astype(jnp.int32))

    base0 = jnp.where(cnt_ge(jnp.int32(0)) >= k, jnp.int32(0),
                      jnp.int32(-(2**31)))

    def body(i, base):
        cand = base | lax.shift_left(jnp.int32(1), 30 - i)
        return jnp.where(cnt_ge(cand) >= k, cand, base)

    big_t = lax.fori_loop(0, 31, body, base0)
    tb = big_t ^ (lax.shift_right_arithmetic(big_t, 31) & jnp.int32(0x7FFFFFFF))
    tval = lax.bitcast_convert_type(tb, jnp.float32)
    gt = loss > tval
    cnt_gt = jnp.sum(gt.astype(jnp.float32))
    sum_gt = jnp.sum(jnp.where(gt, loss, 0.0))
    res = (sum_gt + (jnp.float32(k) - cnt_gt) * tval) / jnp.float32(k)
    out_ref[...] = jnp.full((1, 1), res, jnp.float32)


@jax.jit
def kernel(inputs, targets):
    n, v = inputs.shape
    k = int(0.25 * n)
    t2 = targets.reshape(n, 1).astype(jnp.int32)
    tail_cols = v - _NFULL * _CB
    loss = pl.pallas_call(
        functools.partial(_stream_body, n_rows=n, v_total=v),
        in_specs=[
            pl.BlockSpec((n, 1), lambda: (0, 0)),
            pl.BlockSpec(memory_space=pl.ANY),
        ],
        out_specs=pl.BlockSpec((n, 1), lambda: (0, 0)),
        out_shape=jax.ShapeDtypeStruct((n, 1), jnp.float32),
        scratch_shapes=[
            pltpu.VMEM((_NBUF, n, _CB), jnp.float32),
            pltpu.VMEM((n, tail_cols), jnp.float32),
            pltpu.VMEM((n, 1), jnp.float32),
            pltpu.VMEM((n, 1), jnp.float32),
            pltpu.VMEM((n, 1), jnp.float32),
            pltpu.SemaphoreType.DMA((_NBUF,)),
            pltpu.SemaphoreType.DMA,
        ],
    )(t2, inputs)
    loss8 = loss.reshape(8, n // 8)
    out = pl.pallas_call(
        functools.partial(_topk_body, k=k),
        out_shape=jax.ShapeDtypeStruct((1, 1), jnp.float32),
    )(loss8)
    return out[0, 0]
